# baseline (speedup 1.0000x reference)
import functools

import jax
import jax.numpy as jnp
from jax import lax
from jax.experimental import pallas as pl
from jax.experimental.pallas import tpu as pltpu
from jax.experimental.pallas import tpu_sc as plsc

EPS = 1e-9
N, C, H, W = 8, 3, 512, 512
LANES = 16
NC, NS = 2, 16
NW = NC * NS
ROWS = H // NW
NWIN = W // LANES
SROWS = 24
NSLOT = 9


def _sc_interp(data2d, dis_flat):
    mesh = plsc.VectorSubcoreMesh(core_axis_name="c", subcore_axis_name="s")

    @functools.partial(
        pl.kernel,
        mesh=mesh,
        out_type=jax.ShapeDtypeStruct((C * H, W), jnp.float32),
        compiler_params=pltpu.CompilerParams(needs_layout_passes=False),
        scratch_types=[
            [pltpu.VMEM((SROWS, W), jnp.float32) for _ in range(NSLOT)],
            [pltpu.VMEM((ROWS, W), jnp.float32) for _ in range(2)],
            pltpu.VMEM((LANES,), jnp.float32),
            [pltpu.SemaphoreType.DMA for _ in range(NSLOT)],
            [pltpu.SemaphoreType.DMA for _ in range(2)],
        ],
    )
    def k(data_hbm, dis_hbm, out_hbm, slabs, accs, dis_v, sems, osems):
        wid = lax.axis_index("s") * NC + lax.axis_index("c")
        h0 = wid * ROWS

        pltpu.sync_copy(dis_hbm, dis_v)

        lane = lax.iota(jnp.int32, LANES)
        i0 = jnp.maximum(14 - 2 * lane, 0)
        i1 = jnp.maximum(15 - 2 * lane, 0)
        g0 = plsc.load_gather(dis_v, [i0])
        g1 = plsc.load_gather(dis_v, [i1])
        t = jnp.abs(g0 * g1)
        t = jnp.where(lane < N, t, 0.0)
        t = jnp.where(t <= EPS, 0.0, t)
        ssum = jnp.sum(t)
        deg = jnp.abs(ssum) <= EPS
        denom = jnp.where(deg, 1.0, ssum)
        w = t / denom
        w = jnp.where(jnp.abs(w) <= EPS, 0.0, w)
        degf = jnp.where(deg, 1.0, 0.0)
        onehot = jnp.where(lane == 0, 1.0, 0.0)
        w = w * (1.0 - degf) + onehot * degf
        zeros_f = jnp.zeros((LANES,), jnp.float32)
        wbc = [zeros_f + w[a] for a in range(N)]

        zeros_i = jnp.zeros((LANES,), jnp.int32)
        MAGIC = jnp.float32(12582912.0)
        dvec = dis_v[...]
        ivec = ((dvec + MAGIC) - MAGIC).astype(jnp.int32)
        dus = [ivec[2 * n] for n in range(N)]
        dvs = [ivec[2 * n + 1] for n in range(N)]

        starts = []
        for n in range(N):
            s_raw = jnp.clip(h0 + dus[n], 0, H - ROWS)
            starts.append(jnp.minimum((s_raw // 8) * 8, H - SROWS))

        def fire(u):
            n, c = u % N, u // N
            slot = u % NSLOT
            row0 = pl.multiple_of((n * C + c) * H + starts[n], 8)
            return pltpu.async_copy(data_hbm.at[pl.ds(row0, SROWS)],
                                    slabs[slot], sems[slot])

        handles = [fire(u) for u in range(NSLOT)]
        out_handles = []

        for c in range(C):
            for n in range(N):
                handles[c * N + n].wait()
            acc = accs[c % 2]
            if c >= 2:
                out_handles[c - 2].wait()
            cslabs = [slabs[(c * N + n) % NSLOT] for n in range(N)]

            @plsc.parallel_loop(0, NWIN)
            def win_body(j, cslabs=cslabs, acc=acc):
                w0 = j * LANES
                cas = [jnp.clip(lane + (w0 + dvs[n]), 0, W - 1)
                       for n in range(N)]

                @plsc.parallel_loop(0, ROWS)
                def row_body(r):
                    h = h0 + r
                    v = None
                    for n in range(N):
                        sr = zeros_i + (jnp.clip(h + dus[n], 0, H - 1)
                                        - starts[n])
                        g = plsc.load_gather(cslabs[n], [sr, cas[n]])
                        v = wbc[n] * g if v is None else v + wbc[n] * g
                    acc[r, pl.ds(w0, LANES)] = v

            for n in range(N):
                u = NSLOT + c * N + n
                if u < C * N:
                    handles.append(fire(u))
            base = pl.multiple_of(c * H + h0, 8)
            out_handles.append(
                pltpu.async_copy(acc, out_hbm.at[pl.ds(base, ROWS)],
                                 osems[c % 2]))

        out_handles[1].wait()
        out_handles[2].wait()

    return k(data2d, dis_flat)


def kernel(pixel, cam_xyz, neighbors, dis, data):
    del pixel, cam_xyz, neighbors
    dis_flat = dis.reshape(2 * N)
    data2d = data.reshape(N * C * H, W)
    out2d = _sc_interp(data2d, dis_flat)
    return out2d.reshape(C, H, W)


# speedup vs baseline: 1.3040x; 1.3040x over previous
import functools

import jax
import jax.numpy as jnp
from jax import lax
from jax.experimental import pallas as pl
from jax.experimental.pallas import tpu as pltpu
from jax.experimental.pallas import tpu_sc as plsc

EPS = 1e-9
N, C, H, W = 8, 3, 512, 512
LANES = 16
NC, NS = 2, 16
NW = NC * NS
ROWS = H // NW
NWIN = W // LANES
NSLOT = 12


def _sc_interp(data2d, dis_flat):
    mesh = plsc.VectorSubcoreMesh(core_axis_name="c", subcore_axis_name="s")

    @functools.partial(
        pl.kernel,
        mesh=mesh,
        out_type=jax.ShapeDtypeStruct((C * H, W), jnp.float32),
        compiler_params=pltpu.CompilerParams(needs_layout_passes=False),
        scratch_types=[
            [pltpu.VMEM((ROWS, W), jnp.float32) for _ in range(NSLOT)],
            [pltpu.VMEM((ROWS,), jnp.int32) for _ in range(NSLOT)],
            [pltpu.VMEM((ROWS, W), jnp.float32) for _ in range(2)],
            pltpu.VMEM((LANES,), jnp.float32),
            [pltpu.SemaphoreType.DMA for _ in range(NSLOT)],
            [pltpu.SemaphoreType.DMA for _ in range(2)],
        ],
    )
    def k(data_hbm, dis_hbm, out_hbm, slabs, idxs, accs,
          dis_v, sems, osems):
        wid = lax.axis_index("s") * NC + lax.axis_index("c")
        h0 = wid * ROWS

        pltpu.sync_copy(dis_hbm, dis_v)

        lane = lax.iota(jnp.int32, LANES)
        i0 = jnp.maximum(14 - 2 * lane, 0)
        i1 = jnp.maximum(15 - 2 * lane, 0)
        g0 = plsc.load_gather(dis_v, [i0])
        g1 = plsc.load_gather(dis_v, [i1])
        t = jnp.abs(g0 * g1)
        t = jnp.where(lane < N, t, 0.0)
        t = jnp.where(t <= EPS, 0.0, t)
        ssum = jnp.sum(t)
        deg = jnp.abs(ssum) <= EPS
        denom = jnp.where(deg, 1.0, ssum)
        w = t / denom
        w = jnp.where(jnp.abs(w) <= EPS, 0.0, w)
        degf = jnp.where(deg, 1.0, 0.0)
        onehot = jnp.where(lane == 0, 1.0, 0.0)
        w = w * (1.0 - degf) + onehot * degf
        zeros_f = jnp.zeros((LANES,), jnp.float32)
        wbc = [zeros_f + w[a] for a in range(N)]

        zeros_i = jnp.zeros((LANES,), jnp.int32)
        MAGIC = jnp.float32(12582912.0)
        dvec = dis_v[...]
        ivec = ((dvec + MAGIC) - MAGIC).astype(jnp.int32)
        dus = [ivec[2 * n] for n in range(N)]
        dvs = [ivec[2 * n + 1] for n in range(N)]

        def fire(u):
            n, c = u % N, u // N
            slot = u % NSLOT
            idxs[slot][...] = (jnp.clip(lane + (h0 + dus[n]), 0, H - 1)
                               + (n * C + c) * H)
            return (pltpu.async_copy(data_hbm.at[idxs[slot].at[pl.ds(0, 8)]],
                                     slabs[slot].at[pl.ds(0, 8)], sems[slot]),
                    pltpu.async_copy(data_hbm.at[idxs[slot].at[pl.ds(8, 8)]],
                                     slabs[slot].at[pl.ds(8, 8)], sems[slot]))

        handles = [fire(u) for u in range(NSLOT)]
        out_handles = []

        for c in range(C):
            for n in range(N):
                handles[c * N + n][0].wait()
                handles[c * N + n][1].wait()
            acc = accs[c % 2]
            if c >= 2:
                out_handles[c - 2].wait()
            cslabs = [slabs[(c * N + n) % NSLOT] for n in range(N)]

            @plsc.parallel_loop(0, NWIN)
            def win_body(j, cslabs=cslabs, acc=acc):
                w0 = j * LANES
                cas = [jnp.clip(lane + (w0 + dvs[n]), 0, W - 1)
                       for n in range(N)]

                @plsc.parallel_loop(0, ROWS)
                def row_body(r):
                    sr = zeros_i + r
                    v = wbc[0] * plsc.load_gather(cslabs[0], [sr, cas[0]])
                    for n in range(1, N):
                        v = v + wbc[n] * plsc.load_gather(cslabs[n],
                                                          [sr, cas[n]])
                    acc[r, pl.ds(w0, LANES)] = v

            for n in range(N):
                u = NSLOT + c * N + n
                if u < C * N:
                    handles.append(fire(u))
            base = pl.multiple_of(c * H + h0, 8)
            out_handles.append(
                pltpu.async_copy(acc, out_hbm.at[pl.ds(base, ROWS)],
                                 osems[c % 2]))

        out_handles[1].wait()
        out_handles[2].wait()

    return k(data2d, dis_flat)


def kernel(pixel, cam_xyz, neighbors, dis, data):
    del pixel, cam_xyz, neighbors
    dis_flat = dis.reshape(2 * N)
    data2d = data.reshape(N * C * H, W)
    out2d = _sc_interp(data2d, dis_flat)
    return out2d.reshape(C, H, W)


# speedup vs baseline: 1.3237x; 1.0151x over previous
import functools

import jax
import jax.numpy as jnp
from jax import lax
from jax.experimental import pallas as pl
from jax.experimental.pallas import tpu as pltpu
from jax.experimental.pallas import tpu_sc as plsc

EPS = 1e-9
N, C, H, W = 8, 3, 512, 512
LANES = 16
NC, NS = 2, 16
NW = NC * NS
ROWS = H // NW
NWIN = W // LANES
NSLOT = 12


def _sc_interp(data2d, dis_flat):
    mesh = plsc.VectorSubcoreMesh(core_axis_name="c", subcore_axis_name="s")

    @functools.partial(
        pl.kernel,
        mesh=mesh,
        out_type=jax.ShapeDtypeStruct((C * H, W), jnp.float32),
        compiler_params=pltpu.CompilerParams(needs_layout_passes=False),
        scratch_types=[
            [pltpu.VMEM((ROWS, W), jnp.float32) for _ in range(NSLOT)],
            [pltpu.VMEM((ROWS,), jnp.int32) for _ in range(NSLOT)],
            [pltpu.VMEM((ROWS, W), jnp.float32) for _ in range(2)],
            pltpu.VMEM((LANES,), jnp.float32),
            [pltpu.SemaphoreType.DMA for _ in range(NSLOT)],
            [pltpu.SemaphoreType.DMA for _ in range(2)],
        ],
    )
    def k(data_hbm, dis_hbm, out_hbm, slabs, idxs, accs,
          dis_v, sems, osems):
        wid = lax.axis_index("s") * NC + lax.axis_index("c")
        h0 = wid * ROWS

        pltpu.sync_copy(dis_hbm, dis_v)

        lane = lax.iota(jnp.int32, LANES)
        i0 = jnp.maximum(14 - 2 * lane, 0)
        i1 = jnp.maximum(15 - 2 * lane, 0)
        g0 = plsc.load_gather(dis_v, [i0])
        g1 = plsc.load_gather(dis_v, [i1])
        t = jnp.abs(g0 * g1)
        t = jnp.where(lane < N, t, 0.0)
        t = jnp.where(t <= EPS, 0.0, t)
        ssum = jnp.sum(t)
        deg = jnp.abs(ssum) <= EPS
        denom = jnp.where(deg, 1.0, ssum)
        w = t / denom
        w = jnp.where(jnp.abs(w) <= EPS, 0.0, w)
        degf = jnp.where(deg, 1.0, 0.0)
        onehot = jnp.where(lane == 0, 1.0, 0.0)
        w = w * (1.0 - degf) + onehot * degf
        zeros_f = jnp.zeros((LANES,), jnp.float32)
        wbc = [zeros_f + w[a] for a in range(N)]

        zeros_i = jnp.zeros((LANES,), jnp.int32)
        MAGIC = jnp.float32(12582912.0)
        dvec = dis_v[...]
        ivec = ((dvec + MAGIC) - MAGIC).astype(jnp.int32)
        dus = [ivec[2 * n] for n in range(N)]
        dvs = [ivec[2 * n + 1] for n in range(N)]

        def fire(u):
            n, c = u % N, u // N
            slot = u % NSLOT
            idxs[slot][...] = (jnp.clip(lane + (h0 + dus[n]), 0, H - 1)
                               + (n * C + c) * H)
            return pltpu.async_copy(data_hbm.at[idxs[slot]], slabs[slot],
                                    sems[slot])

        handles = [fire(u) for u in range(NSLOT)]
        out_handles = []

        for c in range(C):
            for n in range(N):
                handles[c * N + n].wait()
            acc = accs[c % 2]
            if c >= 2:
                out_handles[c - 2].wait()
            cslabs = [slabs[(c * N + n) % NSLOT] for n in range(N)]

            @plsc.parallel_loop(0, NWIN)
            def win_body(j, cslabs=cslabs, acc=acc):
                w0 = j * LANES
                cas = [jnp.clip(lane + (w0 + dvs[n]), 0, W - 1)
                       for n in range(N)]

                @plsc.parallel_loop(0, ROWS)
                def row_body(r):
                    sr = zeros_i + r
                    v = wbc[0] * plsc.load_gather(cslabs[0], [sr, cas[0]])
                    for n in range(1, N):
                        v = v + wbc[n] * plsc.load_gather(cslabs[n],
                                                          [sr, cas[n]])
                    acc[r, pl.ds(w0, LANES)] = v

            for n in range(N):
                u = NSLOT + c * N + n
                if u < C * N:
                    handles.append(fire(u))
            base = pl.multiple_of(c * H + h0, 8)
            out_handles.append(
                pltpu.async_copy(acc, out_hbm.at[pl.ds(base, ROWS)],
                                 osems[c % 2]))

        out_handles[1].wait()
        out_handles[2].wait()

    return k(data2d, dis_flat)


def kernel(pixel, cam_xyz, neighbors, dis, data):
    del pixel, cam_xyz, neighbors
    dis_flat = dis.reshape(2 * N)
    data2d = data.reshape(N * C * H, W)
    out2d = _sc_interp(data2d, dis_flat)
    return out2d.reshape(C, H, W)
